# Initial kernel scaffold; baseline (speedup 1.0000x reference)
import jax
import jax.numpy as jnp
from jax.experimental import pallas as pl


def kernel(x, table):
    raise NotImplementedError("write your pallas kernel here")


# speedup vs baseline: 2.4355x; 2.4355x over previous
import functools

import jax
import jax.numpy as jnp
from jax import lax
from jax.experimental import pallas as pl
from jax.experimental.pallas import tpu as pltpu
from jax.experimental.pallas import tpu_sc as plsc

BATCH = 4
SEQ_LEN = 8192
HIDDEN = 768
TOTAL = BATCH * SEQ_LEN
NUM_WORKERS = 32
PER_WORKER = TOTAL // NUM_WORKERS
CHUNK = 64
NCHUNKS = PER_WORKER // CHUNK


def _make_sc_gather():
    mesh = plsc.VectorSubcoreMesh(core_axis_name="c", subcore_axis_name="s")

    @functools.partial(
        pl.kernel,
        mesh=mesh,
        out_type=jax.ShapeDtypeStruct((TOTAL, HIDDEN), jnp.float32),
        scratch_types=[
            pltpu.VMEM((PER_WORKER,), jnp.int32),
            pltpu.VMEM((2, CHUNK, HIDDEN), jnp.float32),
            pltpu.SemaphoreType.DMA,
            pltpu.SemaphoreType.DMA,
        ],
    )
    def sc_gather(table_hbm, idx_hbm, out_hbm, idx_v, rows_v, gsem, ssem):
        wid = lax.axis_index("s") * 2 + lax.axis_index("c")
        base = wid * PER_WORKER
        pltpu.sync_copy(idx_hbm.at[pl.ds(base, PER_WORKER)], idx_v)

        def start_gather(j, slot):
            return pltpu.async_copy(
                table_hbm.at[idx_v.at[pl.ds(j * CHUNK, CHUNK)]],
                rows_v.at[slot],
                gsem,
            )

        def start_store(j, slot):
            return pltpu.async_copy(
                rows_v.at[slot],
                out_hbm.at[pl.ds(base + j * CHUNK, CHUNK)],
                ssem,
            )

        gathers = [None] * NCHUNKS
        stores = [None] * NCHUNKS
        gathers[0] = start_gather(0, 0)
        for j in range(NCHUNKS):
            slot = j % 2
            gathers[j].wait()
            if j >= 1:
                stores[j - 1].wait()
            if j + 1 < NCHUNKS:
                gathers[j + 1] = start_gather(j + 1, 1 - slot)
            stores[j] = start_store(j, slot)
        stores[NCHUNKS - 1].wait()

    return sc_gather


_sc_gather = _make_sc_gather()


@jax.jit
def kernel(x, table):
    out = _sc_gather(table, x.reshape(TOTAL))
    return out.reshape(BATCH, SEQ_LEN, HIDDEN)
